# Initial kernel scaffold; baseline (speedup 1.0000x reference)
import jax
import jax.numpy as jnp
from jax.experimental import pallas as pl


def kernel(x, edge_index, Wl1, Wr1, b1, gamma1, beta1, Wl2, Wr2, b2):
    raise NotImplementedError("write your pallas kernel here")


# speedup vs baseline: 3.2655x; 3.2655x over previous
import functools

import jax
import jax.numpy as jnp
import numpy as np
from jax import lax
from jax.experimental import pallas as pl
from jax.experimental.pallas import tpu as pltpu
from jax.experimental.pallas import tpu_sc as plsc

N = 10000
D = 128
E = 320000

NC = 2
NS = 16
NW = NC * NS
K = 128
C = 80
E_PAD = NW * C * K
N_ACC = 10240
ROWS_PER_TILE = N_ACC // NS
ZCH = 128

BN_SCALE = float(1.0 / np.sqrt(1.0 + 1e-5))


def _sc_edge_pass(with_deg):
  out_type = [jax.ShapeDtypeStruct((NC, N_ACC, D), jnp.float32)]
  if with_deg:
    out_type.append(jax.ShapeDtypeStruct((NC, N_ACC, D), jnp.float32))

  scratch = [
      pltpu.VMEM((2, K), jnp.int32),
      pltpu.VMEM((2, K), jnp.int32),
      pltpu.VMEM((2, K, D), jnp.float32),
      pltpu.VMEM_SHARED((N_ACC, D), jnp.float32),
      pltpu.SemaphoreType.DMA,
      pltpu.SemaphoreType.DMA,
  ]

  def body(*refs):
    if with_deg:
      (table, src, dst, zrows, ones_hbm,
       acc_out, deg_out, src_v, dst_v, rows_v, acc_sh, sem0, sem1) = refs
    else:
      (table, src, dst, zrows,
       acc_out, src_v, dst_v, rows_v, acc_sh, sem0, sem1) = refs
    sems = (sem0, sem1)

    c = lax.axis_index("c")
    s = lax.axis_index("s")
    wid = c * NS + s

    def zero_acc():
      pltpu.sync_copy(zrows, rows_v.at[0])
      for b in range(ROWS_PER_TILE // ZCH):
        pltpu.sync_copy(rows_v.at[0],
                        acc_sh.at[pl.ds(s * ROWS_PER_TILE + b * ZCH, ZCH)])

    def copy_out(dst_hbm):
      for b in range(ROWS_PER_TILE // ZCH):
        r0 = s * ROWS_PER_TILE + b * ZCH
        pltpu.sync_copy(acc_sh.at[pl.ds(r0, ZCH)], rows_v.at[0])
        pltpu.sync_copy(rows_v.at[0], dst_hbm.at[c, pl.ds(r0, ZCH)])

    def stage_and_gather(jj, b):
      pltpu.sync_copy(src.at[wid, jj], src_v.at[b])
      pltpu.sync_copy(dst.at[wid, jj], dst_v.at[b])
      pltpu.async_copy(table.at[src_v.at[b]], rows_v.at[b], sems[b])

    zero_acc()
    plsc.subcore_barrier()

    stage_and_gather(0, 0)

    def pair(j2, carry):
      for b in (0, 1):
        j = j2 * 2 + b
        nb = 1 - b
        pltpu.make_async_copy(table.at[src_v.at[b]], rows_v.at[b],
                              sems[b]).wait()
        jn = jnp.minimum(j + 1, C - 1)
        stage_and_gather(jn, nb)
        pltpu.sync_copy(rows_v.at[b], acc_sh.at[dst_v.at[b]], add=True)
      return carry

    lax.fori_loop(0, C // 2, pair, 0)
    pltpu.make_async_copy(table.at[src_v.at[0]], rows_v.at[0], sems[0]).wait()

    plsc.subcore_barrier()
    copy_out(acc_out)

    if with_deg:
      plsc.subcore_barrier()
      zero_acc()
      plsc.subcore_barrier()
      pltpu.sync_copy(ones_hbm, rows_v.at[1])
      pltpu.sync_copy(dst.at[wid, 0], dst_v.at[0])

      def dpair(j2, carry):
        for b in (0, 1):
          j = j2 * 2 + b
          nb = 1 - b
          jn = jnp.minimum(j + 1, C - 1)
          pltpu.sync_copy(dst.at[wid, jn], dst_v.at[nb])
          pltpu.sync_copy(rows_v.at[1], acc_sh.at[dst_v.at[b]], add=True)
        return carry

      lax.fori_loop(0, C // 2, dpair, 0)
      plsc.subcore_barrier()
      copy_out(deg_out)

  mesh = plsc.VectorSubcoreMesh(core_axis_name="c", subcore_axis_name="s",
                                num_cores=NC, num_subcores=NS)
  return pl.kernel(body, out_type=tuple(out_type), mesh=mesh,
                   scratch_types=scratch)


@functools.cache
def _sc_pass_deg():
  return _sc_edge_pass(with_deg=True)


@functools.cache
def _sc_pass():
  return _sc_edge_pass(with_deg=False)


def _tc_dense1(acc_ref, deg_ref, x_ref, wlT_ref, wrT_ref, b_ref, g_ref,
               be_ref, o_ref):
  deg = deg_ref[0, :, :1] + deg_ref[1, :, :1]
  deg = jnp.maximum(deg, 1.0)
  agg = (acc_ref[0] + acc_ref[1]) / deg
  h = (jnp.dot(agg, wlT_ref[...], preferred_element_type=jnp.float32)
       + jnp.dot(x_ref[...], wrT_ref[...], preferred_element_type=jnp.float32)
       + b_ref[...])
  h = h * (g_ref[...] * BN_SCALE) + be_ref[...]
  o_ref[...] = jnp.maximum(h, 0.0)


def _tc_dense2(acc_ref, deg_ref, x_ref, wlT_ref, wrT_ref, b_ref, o_ref):
  deg = deg_ref[0, :, :1] + deg_ref[1, :, :1]
  deg = jnp.maximum(deg, 1.0)
  agg = (acc_ref[0] + acc_ref[1]) / deg
  o_ref[...] = (jnp.dot(agg, wlT_ref[...], preferred_element_type=jnp.float32)
                + jnp.dot(x_ref[...], wrT_ref[...],
                          preferred_element_type=jnp.float32)
                + b_ref[...])


_TC_R = 256
_TC_GRID = N_ACC // _TC_R

_row_spec = pl.BlockSpec((_TC_R, D), lambda i: (i, 0))
_acc_spec = pl.BlockSpec((NC, _TC_R, D), lambda i: (0, i, 0))
_deg_spec = pl.BlockSpec((NC, _TC_R, D), lambda i: (0, i, 0))
_w_spec = pl.BlockSpec((D, D), lambda i: (0, 0))
_v_spec = pl.BlockSpec((1, D), lambda i: (0, 0))

_dense1_call = pl.pallas_call(
    _tc_dense1,
    grid=(_TC_GRID,),
    in_specs=[_acc_spec, _deg_spec, _row_spec, _w_spec, _w_spec, _v_spec,
              _v_spec, _v_spec],
    out_specs=_row_spec,
    out_shape=jax.ShapeDtypeStruct((N_ACC, D), jnp.float32),
)

_dense2_call = pl.pallas_call(
    _tc_dense2,
    grid=(_TC_GRID,),
    in_specs=[_acc_spec, _deg_spec, _row_spec, _w_spec, _w_spec, _v_spec],
    out_specs=_row_spec,
    out_shape=jax.ShapeDtypeStruct((N_ACC, D), jnp.float32),
)


def kernel(x, edge_index, Wl1, Wr1, b1, gamma1, beta1, Wl2, Wr2, b2):
  src = edge_index[0].astype(jnp.int32)
  dst = edge_index[1].astype(jnp.int32)
  pad = E_PAD - E
  src_p = jnp.concatenate([src, jnp.zeros((pad,), jnp.int32)])
  dst_p = jnp.concatenate([dst, jnp.full((pad,), N, jnp.int32)])
  src3 = src_p.reshape(NW, C, K)
  dst3 = dst_p.reshape(NW, C, K)

  x_pad = jnp.zeros((N_ACC, D), jnp.float32).at[:N].set(x)
  zrows = jnp.zeros((ZCH, D), jnp.float32)
  ones_rows = jnp.ones((K, D), jnp.float32)

  acc1, deg = _sc_pass_deg()(x_pad, src3, dst3, zrows, ones_rows)
  h = _dense1_call(acc1, deg, x_pad, Wl1.T, Wr1.T, b1.reshape(1, D),
                   gamma1.reshape(1, D), beta1.reshape(1, D))
  (acc2,) = _sc_pass()(h, src3, dst3, zrows)
  out = _dense2_call(acc2, deg, h, Wl2.T, Wr2.T, b2.reshape(1, D))
  return out[:N]


# speedup vs baseline: 3.6880x; 1.1294x over previous
import functools

import jax
import jax.numpy as jnp
import numpy as np
from jax import lax
from jax.experimental import pallas as pl
from jax.experimental.pallas import tpu as pltpu
from jax.experimental.pallas import tpu_sc as plsc

N = 10000
D = 128
E = 320000

NC = 2
NS = 16
NW = NC * NS
K = 128
C = 79
E_PAD = NW * C * K
N_ACC = 10240
ROWS_PER_TILE = N_ACC // NS
ZCH = 128

BN_SCALE = float(1.0 / np.sqrt(1.0 + 1e-5))


def _sc_edge_pass(with_deg):
  out_type = [jax.ShapeDtypeStruct((NC, N_ACC, D), jnp.float32)]
  if with_deg:
    out_type.append(jax.ShapeDtypeStruct((NC, N_ACC, D), jnp.float32))

  scratch = [
      pltpu.VMEM((1, K), jnp.int32),
      pltpu.VMEM((1, K), jnp.int32),
      pltpu.VMEM((K, D), jnp.float32),
      pltpu.VMEM_SHARED((N_ACC, D), jnp.float32),
      pltpu.SemaphoreType.DMA,
  ]

  def body(*refs):
    if with_deg:
      (table, src, dst, zrows, ones_hbm,
       acc_out, deg_out, src_v, dst_v, rows_v, acc_sh, sem) = refs
    else:
      table, src, dst, zrows, acc_out, src_v, dst_v, rows_v, acc_sh, sem = refs

    c = lax.axis_index("c")
    s = lax.axis_index("s")
    wid = c * NS + s

    def zero_acc():
      pltpu.sync_copy(zrows, rows_v)
      for b in range(ROWS_PER_TILE // ZCH):
        pltpu.sync_copy(rows_v,
                        acc_sh.at[pl.ds(s * ROWS_PER_TILE + b * ZCH, ZCH)])

    def copy_out(dst_hbm):
      for b in range(ROWS_PER_TILE // ZCH):
        r0 = s * ROWS_PER_TILE + b * ZCH
        pltpu.sync_copy(acc_sh.at[pl.ds(r0, ZCH)], rows_v)
        pltpu.sync_copy(rows_v, dst_hbm.at[c, pl.ds(r0, ZCH)])

    zero_acc()
    plsc.subcore_barrier()

    def chunk(j, carry):
      pltpu.sync_copy(src.at[wid, j], src_v.at[0])
      pltpu.sync_copy(dst.at[wid, j], dst_v.at[0])
      pltpu.async_copy(table.at[src_v.at[0]], rows_v, sem).wait()
      pltpu.sync_copy(rows_v, acc_sh.at[dst_v.at[0]], add=True)
      return carry

    lax.fori_loop(0, C, chunk, 0)
    plsc.subcore_barrier()
    copy_out(acc_out)

    if with_deg:
      plsc.subcore_barrier()
      zero_acc()
      plsc.subcore_barrier()
      pltpu.sync_copy(ones_hbm, rows_v)

      def dchunk(j, carry):
        pltpu.sync_copy(dst.at[wid, j], dst_v.at[0])
        pltpu.sync_copy(rows_v, acc_sh.at[dst_v.at[0]], add=True)
        return carry

      lax.fori_loop(0, C, dchunk, 0)
      plsc.subcore_barrier()
      copy_out(deg_out)

  mesh = plsc.VectorSubcoreMesh(core_axis_name="c", subcore_axis_name="s",
                                num_cores=NC, num_subcores=NS)
  return pl.kernel(body, out_type=tuple(out_type), mesh=mesh,
                   scratch_types=scratch)


@functools.cache
def _sc_pass_deg():
  return _sc_edge_pass(with_deg=True)


@functools.cache
def _sc_pass():
  return _sc_edge_pass(with_deg=False)


def _tc_dense1(acc_ref, deg_ref, x_ref, wlT_ref, wrT_ref, b_ref, g_ref,
               be_ref, o_ref):
  deg = deg_ref[0, :, :1] + deg_ref[1, :, :1]
  deg = jnp.maximum(deg, 1.0)
  agg = (acc_ref[0] + acc_ref[1]) / deg
  h = (jnp.dot(agg, wlT_ref[...], preferred_element_type=jnp.float32)
       + jnp.dot(x_ref[...], wrT_ref[...], preferred_element_type=jnp.float32)
       + b_ref[...])
  h = h * (g_ref[...] * BN_SCALE) + be_ref[...]
  o_ref[...] = jnp.maximum(h, 0.0)


def _tc_dense2(acc_ref, deg_ref, x_ref, wlT_ref, wrT_ref, b_ref, o_ref):
  deg = deg_ref[0, :, :1] + deg_ref[1, :, :1]
  deg = jnp.maximum(deg, 1.0)
  agg = (acc_ref[0] + acc_ref[1]) / deg
  o_ref[...] = (jnp.dot(agg, wlT_ref[...], preferred_element_type=jnp.float32)
                + jnp.dot(x_ref[...], wrT_ref[...],
                          preferred_element_type=jnp.float32)
                + b_ref[...])


_TC_R = 256
_TC_GRID = N_ACC // _TC_R

_row_spec = pl.BlockSpec((_TC_R, D), lambda i: (i, 0))
_acc_spec = pl.BlockSpec((NC, _TC_R, D), lambda i: (0, i, 0))
_deg_spec = pl.BlockSpec((NC, _TC_R, D), lambda i: (0, i, 0))
_w_spec = pl.BlockSpec((D, D), lambda i: (0, 0))
_v_spec = pl.BlockSpec((1, D), lambda i: (0, 0))

_dense1_call = pl.pallas_call(
    _tc_dense1,
    grid=(_TC_GRID,),
    in_specs=[_acc_spec, _deg_spec, _row_spec, _w_spec, _w_spec, _v_spec,
              _v_spec, _v_spec],
    out_specs=_row_spec,
    out_shape=jax.ShapeDtypeStruct((N_ACC, D), jnp.float32),
)

_dense2_call = pl.pallas_call(
    _tc_dense2,
    grid=(_TC_GRID,),
    in_specs=[_acc_spec, _deg_spec, _row_spec, _w_spec, _w_spec, _v_spec],
    out_specs=_row_spec,
    out_shape=jax.ShapeDtypeStruct((N_ACC, D), jnp.float32),
)


def kernel(x, edge_index, Wl1, Wr1, b1, gamma1, beta1, Wl2, Wr2, b2):
  src = edge_index[0].astype(jnp.int32)
  dst = edge_index[1].astype(jnp.int32)
  pad = E_PAD - E
  src_p = jnp.concatenate([src, jnp.zeros((pad,), jnp.int32)])
  dst_p = jnp.concatenate([dst, jnp.full((pad,), N, jnp.int32)])
  src3 = src_p.reshape(NW, C, K)
  dst3 = dst_p.reshape(NW, C, K)

  x_pad = jnp.zeros((N_ACC, D), jnp.float32).at[:N].set(x)
  zrows = jnp.zeros((ZCH, D), jnp.float32)
  ones_rows = jnp.ones((K, D), jnp.float32)

  acc1, deg = _sc_pass_deg()(x_pad, src3, dst3, zrows, ones_rows)
  h = _dense1_call(acc1, deg, x_pad, Wl1.T, Wr1.T, b1.reshape(1, D),
                   gamma1.reshape(1, D), beta1.reshape(1, D))
  (acc2,) = _sc_pass()(h, src3, dst3, zrows)
  out = _dense2_call(acc2, deg, h, Wl2.T, Wr2.T, b2.reshape(1, D))
  return out[:N]
